# Initial kernel scaffold; baseline (speedup 1.0000x reference)
import jax
import jax.numpy as jnp
from jax.experimental import pallas as pl


def kernel(input, target, mask, Sim_Matrix):
    raise NotImplementedError("write your pallas kernel here")


# speedup vs baseline: 1.6614x; 1.6614x over previous
import functools

import jax
import jax.numpy as jnp
from jax import lax
from jax.experimental import pallas as pl
from jax.experimental.pallas import tpu as pltpu
from jax.experimental.pallas import tpu_sc as plsc

ALPHA = 0.7
TAU_WORD = 0.1

NC = 2
NS = 16
L = 16
NW = NC * NS
UNROLL = 8
LANE_TILE = 128


@functools.lru_cache(maxsize=None)
def _make_sc_call(N, V):
    v_main = (V // LANE_TILE) * LANE_TILE
    v_tail = V - v_main
    assert N % (2 * NW) == 0 and v_main % (L * UNROLL) == 0
    assert v_tail == L
    tok_per_w = N // NW
    n_chunk = v_main // (L * UNROLL)
    mesh = plsc.VectorSubcoreMesh(core_axis_name="c", subcore_axis_name="s")

    @functools.partial(
        pl.kernel,
        out_type=jax.ShapeDtypeStruct((NW, 4 * L), jnp.float32),
        mesh=mesh,
        compiler_params=pltpu.CompilerParams(needs_layout_passes=False),
        scratch_types=[
            pltpu.VMEM((8 * N,), jnp.int32),
            pltpu.VMEM((N,), jnp.float32),
            pltpu.VMEM((1, V), jnp.float32),
            pltpu.VMEM((1, V), jnp.float32),
            pltpu.VMEM((1, v_main), jnp.float32),
            pltpu.VMEM((1, v_main), jnp.float32),
            pltpu.VMEM((1, LANE_TILE), jnp.float32),
            pltpu.VMEM((1, LANE_TILE), jnp.float32),
            pltpu.VMEM((4 * L,), jnp.float32),
            pltpu.SemaphoreType.DMA,
            pltpu.SemaphoreType.DMA,
        ],
    )
    def sc_call(in_hbm, tgt_hbm, msk_hbm, sim_hbm, tail_hbm, out_hbm,
                tgt_v, msk_v, in_v0, in_v1, sim_v0, sim_v1, tl_v0, tl_v1,
                res_v, sem0, sem1):
        wid = lax.axis_index("s") * NC + lax.axis_index("c")
        base = wid * tok_per_w
        pltpu.sync_copy(tgt_hbm, tgt_v)
        pltpu.sync_copy(msk_hbm, msk_v)
        lane = lax.iota(jnp.int32, L)
        zeros_i = jnp.zeros((L,), jnp.int32)

        def fire(t, in_buf, sim_buf, tl_buf, sem):
            pltpu.async_copy(in_hbm.at[pl.ds(t, 1)], in_buf, sem)
            idx = tgt_v.at[pl.ds(8 * t, 1)]
            pltpu.async_copy(sim_hbm.at[idx, pl.ds(0, v_main)], sim_buf, sem)
            pltpu.async_copy(tail_hbm.at[idx], tl_buf, sem)

        def wait(t, in_buf, sim_buf, tl_buf, sem):
            idx = tgt_v.at[pl.ds(8 * t, 1)]
            pltpu.make_async_copy(in_hbm.at[pl.ds(t, 1)], in_buf, sem).wait()
            pltpu.make_async_copy(
                sim_hbm.at[idx, pl.ds(0, v_main)], sim_buf, sem).wait()
            pltpu.make_async_copy(tail_hbm.at[idx], tl_buf, sem).wait()

        def compute(t, in_buf, sim_buf, tl_buf, accs):
            acc_n, acc_d, acc_ml, acc_m = accs

            def inner(k, c):
                n, d = c
                off = k * (L * UNROLL)
                for u in range(UNROLL):
                    vs = sim_buf[0, pl.ds(off + u * L, L)]
                    vi = in_buf[0, pl.ds(off + u * L, L)]
                    e = jnp.exp(vs * (1.0 / TAU_WORD) - (1.0 / TAU_WORD))
                    n = n + vi * e
                    d = d + e
                return (n, d)

            zf = jnp.zeros((L,), jnp.float32)
            tn, td = lax.fori_loop(0, n_chunk, inner, (zf, zf))
            vs = tl_buf[0, pl.ds(0, L)]
            vi = in_buf[0, pl.ds(v_main, L)]
            e = jnp.exp(vs * (1.0 / TAU_WORD) - (1.0 / TAU_WORD))
            tn = tn + vi * e
            td = td + e
            tv = jnp.full((L,), t, jnp.int32)
            mv = plsc.load_gather(msk_v, [tv])
            tgt_vec = plsc.load_gather(tgt_v, [tv * 8])
            g = plsc.load_gather(in_buf, [zeros_i, tgt_vec])
            lane0 = lane == 0
            return (acc_n + tn * mv,
                    acc_d + td * mv,
                    acc_ml + jnp.where(lane0, g * mv, 0.0),
                    acc_m + jnp.where(lane0, mv, 0.0))

        zf = jnp.zeros((L,), jnp.float32)
        fire(base, in_v0, sim_v0, tl_v0, sem0)

        def pair(k, accs):
            t0 = base + 2 * k
            t1 = t0 + 1
            fire(t1, in_v1, sim_v1, tl_v1, sem1)
            wait(t0, in_v0, sim_v0, tl_v0, sem0)
            accs = compute(t0, in_v0, sim_v0, tl_v0, accs)

            @pl.when(2 * k + 2 < tok_per_w)
            def _():
                fire(t0 + 2, in_v0, sim_v0, tl_v0, sem0)

            wait(t1, in_v1, sim_v1, tl_v1, sem1)
            accs = compute(t1, in_v1, sim_v1, tl_v1, accs)
            return accs

        acc_n, acc_d, acc_ml, acc_m = lax.fori_loop(
            0, tok_per_w // 2, pair, (zf, zf, zf, zf))
        res_v[pl.ds(0, L)] = acc_n
        res_v[pl.ds(L, L)] = acc_d
        res_v[pl.ds(2 * L, L)] = acc_ml
        res_v[pl.ds(3 * L, L)] = acc_m
        pltpu.sync_copy(res_v, out_hbm.at[wid])

    return sc_call


def kernel(input, target, mask, Sim_Matrix):
    b, t, v = input.shape
    flat_t = target[:, :t].reshape(-1).astype(jnp.int32)
    n = flat_t.shape[0]
    tgt8 = jnp.broadcast_to(flat_t[:, None], (n, 8)).reshape(-1)
    flat_m = mask[:, :t].reshape(-1).astype(jnp.float32)
    flat_in = input.reshape(-1, v)
    v_main = (v // LANE_TILE) * LANE_TILE
    sim_tail = jnp.pad(Sim_Matrix[:, v_main:],
                       ((0, 0), (0, LANE_TILE - (v - v_main))))
    partials = _make_sc_call(n, v)(
        flat_in, tgt8, flat_m, Sim_Matrix, sim_tail)
    p = partials.reshape(NW, 4, L)
    num = jnp.sum(p[:, 0, :])
    den = jnp.sum(p[:, 1, :])
    ml_sum = jnp.sum(p[:, 2, :])
    m_sum = jnp.sum(p[:, 3, :])
    ml_output = -ml_sum / m_sum
    smooth_loss = -num / den
    total = ALPHA * smooth_loss + (1.0 - ALPHA) * ml_output
    return (ml_output, total)


# speedup vs baseline: 1.6624x; 1.0006x over previous
import functools

import jax
import jax.numpy as jnp
from jax import lax
from jax.experimental import pallas as pl
from jax.experimental.pallas import tpu as pltpu
from jax.experimental.pallas import tpu_sc as plsc

ALPHA = 0.7
TAU_WORD = 0.1

NC = 2
NS = 16
L = 16
NW = NC * NS
UNROLL = 8
LANE_TILE = 128


@functools.lru_cache(maxsize=None)
def _make_sc_call(N, V):
    v_main = (V // LANE_TILE) * LANE_TILE
    v_tail = V - v_main
    assert N % (2 * NW) == 0 and v_main % (L * UNROLL) == 0
    assert v_tail == L
    tok_per_w = N // NW
    n_chunk = v_main // (L * UNROLL)
    mesh = plsc.VectorSubcoreMesh(core_axis_name="c", subcore_axis_name="s")

    @functools.partial(
        pl.kernel,
        out_type=jax.ShapeDtypeStruct((NW, 4 * L), jnp.float32),
        mesh=mesh,
        compiler_params=pltpu.CompilerParams(needs_layout_passes=False),
        scratch_types=[
            pltpu.VMEM((8 * N,), jnp.int32),
            pltpu.VMEM((N,), jnp.float32),
            pltpu.VMEM((1, V), jnp.float32),
            pltpu.VMEM((1, V), jnp.float32),
            pltpu.VMEM((1, v_main), jnp.float32),
            pltpu.VMEM((1, v_main), jnp.float32),
            pltpu.VMEM((1, LANE_TILE), jnp.float32),
            pltpu.VMEM((1, LANE_TILE), jnp.float32),
            pltpu.VMEM((4 * L,), jnp.float32),
            pltpu.SemaphoreType.DMA,
            pltpu.SemaphoreType.DMA,
        ],
    )
    def sc_call(in_hbm, tgt_hbm, msk_hbm, sim_hbm, tail_hbm, out_hbm,
                tgt_v, msk_v, in_v0, in_v1, sim_v0, sim_v1, tl_v0, tl_v1,
                res_v, sem0, sem1):
        wid = lax.axis_index("s") * NC + lax.axis_index("c")
        base = wid * tok_per_w
        pltpu.sync_copy(tgt_hbm, tgt_v)
        pltpu.sync_copy(msk_hbm, msk_v)
        lane = lax.iota(jnp.int32, L)
        zeros_i = jnp.zeros((L,), jnp.int32)

        def fire(t, in_buf, sim_buf, tl_buf, sem):
            pltpu.async_copy(in_hbm.at[pl.ds(t, 1)], in_buf, sem)
            idx = tgt_v.at[pl.ds(8 * t, 1)]
            pltpu.async_copy(sim_hbm.at[idx, pl.ds(0, v_main)], sim_buf, sem)
            pltpu.async_copy(tail_hbm.at[idx], tl_buf, sem)

        def wait(t, in_buf, sim_buf, tl_buf, sem):
            idx = tgt_v.at[pl.ds(8 * t, 1)]
            pltpu.make_async_copy(in_hbm.at[pl.ds(t, 1)], in_buf, sem).wait()
            pltpu.make_async_copy(
                sim_hbm.at[idx, pl.ds(0, v_main)], sim_buf, sem).wait()
            pltpu.make_async_copy(tail_hbm.at[idx], tl_buf, sem).wait()

        def compute(t, in_buf, sim_buf, tl_buf, accs):
            acc_n, acc_d, acc_ml, acc_m = accs

            def inner(k, c):
                ns, ds = c
                off = k * (L * UNROLL)
                ns_out, ds_out = [], []
                for u in range(UNROLL):
                    vs = sim_buf[0, pl.ds(off + u * L, L)]
                    vi = in_buf[0, pl.ds(off + u * L, L)]
                    e = jnp.exp(vs * (1.0 / TAU_WORD) - (1.0 / TAU_WORD))
                    ns_out.append(ns[u] + vi * e)
                    ds_out.append(ds[u] + e)
                return (tuple(ns_out), tuple(ds_out))

            zf = jnp.zeros((L,), jnp.float32)
            zs = (zf,) * UNROLL
            ns, ds = lax.fori_loop(0, n_chunk, inner, (zs, zs))
            tn = functools.reduce(lambda a, b: a + b, ns)
            td = functools.reduce(lambda a, b: a + b, ds)
            vs = tl_buf[0, pl.ds(0, L)]
            vi = in_buf[0, pl.ds(v_main, L)]
            e = jnp.exp(vs * (1.0 / TAU_WORD) - (1.0 / TAU_WORD))
            tn = tn + vi * e
            td = td + e
            tv = jnp.full((L,), t, jnp.int32)
            mv = plsc.load_gather(msk_v, [tv])
            tgt_vec = plsc.load_gather(tgt_v, [tv * 8])
            g = plsc.load_gather(in_buf, [zeros_i, tgt_vec])
            lane0 = lane == 0
            return (acc_n + tn * mv,
                    acc_d + td * mv,
                    acc_ml + jnp.where(lane0, g * mv, 0.0),
                    acc_m + jnp.where(lane0, mv, 0.0))

        zf = jnp.zeros((L,), jnp.float32)
        fire(base, in_v0, sim_v0, tl_v0, sem0)

        def pair(k, accs):
            t0 = base + 2 * k
            t1 = t0 + 1
            fire(t1, in_v1, sim_v1, tl_v1, sem1)
            wait(t0, in_v0, sim_v0, tl_v0, sem0)
            accs = compute(t0, in_v0, sim_v0, tl_v0, accs)

            @pl.when(2 * k + 2 < tok_per_w)
            def _():
                fire(t0 + 2, in_v0, sim_v0, tl_v0, sem0)

            wait(t1, in_v1, sim_v1, tl_v1, sem1)
            accs = compute(t1, in_v1, sim_v1, tl_v1, accs)
            return accs

        acc_n, acc_d, acc_ml, acc_m = lax.fori_loop(
            0, tok_per_w // 2, pair, (zf, zf, zf, zf))
        res_v[pl.ds(0, L)] = acc_n
        res_v[pl.ds(L, L)] = acc_d
        res_v[pl.ds(2 * L, L)] = acc_ml
        res_v[pl.ds(3 * L, L)] = acc_m
        pltpu.sync_copy(res_v, out_hbm.at[wid])

    return sc_call


def kernel(input, target, mask, Sim_Matrix):
    b, t, v = input.shape
    flat_t = target[:, :t].reshape(-1).astype(jnp.int32)
    n = flat_t.shape[0]
    tgt8 = jnp.broadcast_to(flat_t[:, None], (n, 8)).reshape(-1)
    flat_m = mask[:, :t].reshape(-1).astype(jnp.float32)
    flat_in = input.reshape(-1, v)
    v_main = (v // LANE_TILE) * LANE_TILE
    sim_tail = jnp.pad(Sim_Matrix[:, v_main:],
                       ((0, 0), (0, LANE_TILE - (v - v_main))))
    partials = _make_sc_call(n, v)(
        flat_in, tgt8, flat_m, Sim_Matrix, sim_tail)
    p = partials.reshape(NW, 4, L)
    num = jnp.sum(p[:, 0, :])
    den = jnp.sum(p[:, 1, :])
    ml_sum = jnp.sum(p[:, 2, :])
    m_sum = jnp.sum(p[:, 3, :])
    ml_output = -ml_sum / m_sum
    smooth_loss = -num / den
    total = ALPHA * smooth_loss + (1.0 - ALPHA) * ml_output
    return (ml_output, total)


# speedup vs baseline: 2.6080x; 1.5688x over previous
import functools

import jax
import jax.numpy as jnp
from jax import lax
from jax.experimental import pallas as pl
from jax.experimental.pallas import tpu as pltpu
from jax.experimental.pallas import tpu_sc as plsc

ALPHA = 0.7
TAU_WORD = 0.1

NC = 2
NS = 16
L = 16
NW = NC * NS
UNROLL = 8
LANE_TILE = 128


@functools.lru_cache(maxsize=None)
def _make_sc_call(B, T, V):
    N = B * T
    v_main = (V // LANE_TILE) * LANE_TILE
    v_tail = V - v_main
    assert N % (2 * NW) == 0 and v_main % (L * UNROLL) == 0
    assert v_tail == L
    tok_per_w = N // NW
    n_chunk = v_main // (L * UNROLL)
    mesh = plsc.VectorSubcoreMesh(core_axis_name="c", subcore_axis_name="s")

    @functools.partial(
        pl.kernel,
        out_type=jax.ShapeDtypeStruct((NW, 4 * L), jnp.float32),
        mesh=mesh,
        compiler_params=pltpu.CompilerParams(needs_layout_passes=False),
        scratch_types=[
            pltpu.VMEM((8 * N,), jnp.int32),
            pltpu.VMEM((N,), jnp.float32),
            pltpu.VMEM((1, V), jnp.float32),
            pltpu.VMEM((1, V), jnp.float32),
            pltpu.VMEM((1, v_main), jnp.float32),
            pltpu.VMEM((1, v_main), jnp.float32),
            pltpu.VMEM((1, LANE_TILE), jnp.float32),
            pltpu.VMEM((1, LANE_TILE), jnp.float32),
            pltpu.VMEM((4 * L,), jnp.float32),
            pltpu.SemaphoreType.DMA,
            pltpu.SemaphoreType.DMA,
        ],
    )
    def sc_call(in_hbm, tgt_hbm, msk_hbm, sim_hbm, tail_hbm, out_hbm,
                tgt_v, msk_v, in_v0, in_v1, sim_v0, sim_v1, tl_v0, tl_v1,
                res_v, sem0, sem1):
        wid = lax.axis_index("s") * NC + lax.axis_index("c")
        base = wid * tok_per_w
        pltpu.sync_copy(tgt_hbm, tgt_v)
        pltpu.sync_copy(msk_hbm, msk_v)
        lane = lax.iota(jnp.int32, L)
        zeros_i = jnp.zeros((L,), jnp.int32)

        def fire(t, in_buf, sim_buf, tl_buf, sem):
            bi = t // T
            pltpu.async_copy(
                in_hbm.at[bi, pl.ds(t - bi * T, 1)], in_buf, sem)
            idx = tgt_v.at[pl.ds(8 * t, 1)]
            pltpu.async_copy(sim_hbm.at[idx, pl.ds(0, v_main)], sim_buf, sem)
            pltpu.async_copy(tail_hbm.at[idx], tl_buf, sem)

        def wait(t, in_buf, sim_buf, tl_buf, sem):
            bi = t // T
            idx = tgt_v.at[pl.ds(8 * t, 1)]
            pltpu.make_async_copy(
                in_hbm.at[bi, pl.ds(t - bi * T, 1)], in_buf, sem).wait()
            pltpu.make_async_copy(
                sim_hbm.at[idx, pl.ds(0, v_main)], sim_buf, sem).wait()
            pltpu.make_async_copy(tail_hbm.at[idx], tl_buf, sem).wait()

        def compute(t, in_buf, sim_buf, tl_buf, accs):
            acc_n, acc_d, acc_ml, acc_m = accs

            def inner(k, c):
                ns, ds = c
                off = k * (L * UNROLL)
                ns_out, ds_out = [], []
                for u in range(UNROLL):
                    vs = sim_buf[0, pl.ds(off + u * L, L)]
                    vi = in_buf[0, pl.ds(off + u * L, L)]
                    e = jnp.exp(vs * (1.0 / TAU_WORD) - (1.0 / TAU_WORD))
                    ns_out.append(ns[u] + vi * e)
                    ds_out.append(ds[u] + e)
                return (tuple(ns_out), tuple(ds_out))

            zf = jnp.zeros((L,), jnp.float32)
            zs = (zf,) * UNROLL
            ns, ds = lax.fori_loop(0, n_chunk, inner, (zs, zs))
            tn = functools.reduce(lambda a, b: a + b, ns)
            td = functools.reduce(lambda a, b: a + b, ds)
            vs = tl_buf[0, pl.ds(0, L)]
            vi = in_buf[0, pl.ds(v_main, L)]
            e = jnp.exp(vs * (1.0 / TAU_WORD) - (1.0 / TAU_WORD))
            tn = tn + vi * e
            td = td + e
            tv = jnp.full((L,), t, jnp.int32)
            mv = plsc.load_gather(msk_v, [tv])
            tgt_vec = plsc.load_gather(tgt_v, [tv * 8])
            g = plsc.load_gather(in_buf, [zeros_i, tgt_vec])
            lane0 = lane == 0
            return (acc_n + tn * mv,
                    acc_d + td * mv,
                    acc_ml + jnp.where(lane0, g * mv, 0.0),
                    acc_m + jnp.where(lane0, mv, 0.0))

        zf = jnp.zeros((L,), jnp.float32)
        fire(base, in_v0, sim_v0, tl_v0, sem0)

        def pair(k, accs):
            t0 = base + 2 * k
            t1 = t0 + 1
            fire(t1, in_v1, sim_v1, tl_v1, sem1)
            wait(t0, in_v0, sim_v0, tl_v0, sem0)
            accs = compute(t0, in_v0, sim_v0, tl_v0, accs)

            @pl.when(2 * k + 2 < tok_per_w)
            def _():
                fire(t0 + 2, in_v0, sim_v0, tl_v0, sem0)

            wait(t1, in_v1, sim_v1, tl_v1, sem1)
            accs = compute(t1, in_v1, sim_v1, tl_v1, accs)
            return accs

        acc_n, acc_d, acc_ml, acc_m = lax.fori_loop(
            0, tok_per_w // 2, pair, (zf, zf, zf, zf))
        res_v[pl.ds(0, L)] = acc_n
        res_v[pl.ds(L, L)] = acc_d
        res_v[pl.ds(2 * L, L)] = acc_ml
        res_v[pl.ds(3 * L, L)] = acc_m
        pltpu.sync_copy(res_v, out_hbm.at[wid])

    return sc_call


def kernel(input, target, mask, Sim_Matrix):
    b, t, v = input.shape
    flat_t = target[:, :t].reshape(-1).astype(jnp.int32)
    n = flat_t.shape[0]
    tgt8 = jnp.broadcast_to(flat_t[:, None], (n, 8)).reshape(-1)
    flat_m = mask[:, :t].reshape(-1).astype(jnp.float32)
    v_main = (v // LANE_TILE) * LANE_TILE
    sim_tail = jnp.pad(Sim_Matrix[:, v_main:],
                       ((0, 0), (0, LANE_TILE - (v - v_main))))
    partials = _make_sc_call(b, t, v)(
        input, tgt8, flat_m, Sim_Matrix, sim_tail)
    p = partials.reshape(NW, 4, L)
    num = jnp.sum(p[:, 0, :])
    den = jnp.sum(p[:, 1, :])
    ml_sum = jnp.sum(p[:, 2, :])
    m_sum = jnp.sum(p[:, 3, :])
    ml_output = -ml_sum / m_sum
    smooth_loss = -num / den
    total = ALPHA * smooth_loss + (1.0 - ALPHA) * ml_output
    return (ml_output, total)
